# baseline (speedup 1.0000x reference)
import functools

import jax
import jax.numpy as jnp
from jax import lax
from jax.experimental import pallas as pl
from jax.experimental.pallas import tpu as pltpu
from jax.experimental.pallas import tpu_sc as plsc

N_ATTRS = 26
VOCAB = 1000
D_MODEL = 128
BATCH = 16384
ROWS = BATCH * N_ATTRS

NC = 2
NS = 16
NW = NC * NS
RPW = ROWS // NW
CHUNK = 128
NCH = RPW // CHUNK


def _fuse_body(prior_ref, opt_ref, attr_ref, out_ref):
    i = pl.program_id(0)
    out_ref[...] = (opt_ref[...] + attr_ref[...]) * prior_ref[i, 0]


def _fused_table(attr_emb, option_embs, prior):
    return pl.pallas_call(
        _fuse_body,
        grid=(N_ATTRS,),
        in_specs=[
            pl.BlockSpec(memory_space=pltpu.SMEM),
            pl.BlockSpec((1, VOCAB, D_MODEL), lambda i: (i, 0, 0)),
            pl.BlockSpec((1, 1, D_MODEL), lambda i: (i, 0, 0)),
        ],
        out_specs=pl.BlockSpec((1, VOCAB, D_MODEL), lambda i: (i, 0, 0)),
        out_shape=jax.ShapeDtypeStruct((N_ATTRS, VOCAB, D_MODEL), jnp.float32),
    )(prior, option_embs, attr_emb.reshape(N_ATTRS, 1, D_MODEL))


PAD = 32


def _idx_body(x_ref, out_ref):
    col = lax.broadcasted_iota(jnp.int32, (BATCH, PAD), 1)
    xpad = jnp.concatenate(
        [x_ref[...], jnp.zeros((BATCH, PAD - N_ATTRS), jnp.int32)], axis=1)
    out_ref[...] = jnp.where(col < N_ATTRS, xpad + col * VOCAB, 0)


def _flat_idx(x):
    return pl.pallas_call(
        _idx_body,
        out_shape=jax.ShapeDtypeStruct((BATCH, PAD), jnp.int32),
    )(x)


BPW = BATCH // NW
CB = 8
CROWS = CB * PAD
GROWS = 128
GPC = CROWS // GROWS
NCHUNK = BPW // CB
IPW = BPW * PAD

_mesh = plsc.VectorSubcoreMesh(core_axis_name="c", subcore_axis_name="s")


@functools.partial(
    pl.kernel,
    mesh=_mesh,
    out_type=jax.ShapeDtypeStruct((BATCH, PAD, D_MODEL), jnp.float32),
    scratch_types=[
        pltpu.VMEM((IPW,), jnp.int32),
        pltpu.VMEM((CROWS, D_MODEL), jnp.float32),
        pltpu.VMEM((CROWS, D_MODEL), jnp.float32),
        pltpu.SemaphoreType.DMA,
        pltpu.SemaphoreType.DMA,
        pltpu.SemaphoreType.DMA,
        pltpu.SemaphoreType.DMA,
    ],
)
def _gather_kernel(table_hbm, idx_hbm, out_hbm, idx_v, buf0, buf1,
                   g0, g1, s0, s1):
    wid = lax.axis_index("s") * NC + lax.axis_index("c")
    bbase = wid * BPW
    pltpu.sync_copy(idx_hbm.at[pl.ds(bbase * PAD, IPW)], idx_v)

    def start_gathers(c, buf, sem):
        for g in range(GPC):
            pltpu.async_copy(
                table_hbm.at[idx_v.at[pl.ds(c * CROWS + g * GROWS, GROWS)]],
                buf.at[pl.ds(g * GROWS, GROWS)], sem)

    def wait_gathers(buf, sem):
        pltpu.make_async_copy(table_hbm.at[pl.ds(0, CROWS)], buf, sem).wait()

    def start_put(c, buf, sem):
        pltpu.async_copy(buf.reshape(CB, PAD, D_MODEL),
                         out_hbm.at[pl.ds(bbase + c * CB, CB)], sem)

    def wait_put(c, buf, sem):
        pltpu.make_async_copy(buf.reshape(CB, PAD, D_MODEL),
                              out_hbm.at[pl.ds(bbase + c * CB, CB)],
                              sem).wait()

    start_gathers(0, buf0, g0)
    start_gathers(1, buf1, g1)

    def body(p, carry):
        c = 2 * p
        wait_gathers(buf0, g0)
        start_put(c, buf0, s0)
        wait_put(c, buf0, s0)
        start_gathers(c + 2, buf0, g0)
        wait_gathers(buf1, g1)
        start_put(c + 1, buf1, s1)
        wait_put(c + 1, buf1, s1)
        start_gathers(c + 3, buf1, g1)
        return carry

    lax.fori_loop(0, NCHUNK // 2 - 1, body, 0)

    c_last = NCHUNK - 2
    wait_gathers(buf0, g0)
    start_put(c_last, buf0, s0)
    wait_gathers(buf1, g1)
    start_put(c_last + 1, buf1, s1)
    wait_put(c_last, buf0, s0)
    wait_put(c_last + 1, buf1, s1)


def kernel(x, attr_emb, option_embs, prior):
    x = x.astype(jnp.int32)
    table = _fused_table(attr_emb, option_embs, prior)
    idx = _flat_idx(x).reshape(BATCH * PAD)
    out = _gather_kernel(table.reshape(N_ATTRS * VOCAB, D_MODEL), idx)
    return out[:, :N_ATTRS, :]


# speedup vs baseline: 7.1079x; 7.1079x over previous
import functools

import jax
import jax.numpy as jnp
from jax import lax
from jax.experimental import pallas as pl
from jax.experimental.pallas import tpu as pltpu
from jax.experimental.pallas import tpu_sc as plsc

N_ATTRS = 26
VOCAB = 1000
D_MODEL = 128
BATCH = 16384
ROWS = BATCH * N_ATTRS

NC = 2
NS = 16
NW = NC * NS
RPW = ROWS // NW
CHUNK = 128
NCH = RPW // CHUNK


def _fuse_body(prior_ref, opt_ref, attr_ref, out_ref):
    i = pl.program_id(0)
    out_ref[...] = (opt_ref[...] + attr_ref[...]) * prior_ref[i, 0]


def _fused_table(attr_emb, option_embs, prior):
    return pl.pallas_call(
        _fuse_body,
        grid=(N_ATTRS,),
        in_specs=[
            pl.BlockSpec(memory_space=pltpu.SMEM),
            pl.BlockSpec((1, VOCAB, D_MODEL), lambda i: (i, 0, 0)),
            pl.BlockSpec((1, 1, D_MODEL), lambda i: (i, 0, 0)),
        ],
        out_specs=pl.BlockSpec((1, VOCAB, D_MODEL), lambda i: (i, 0, 0)),
        out_shape=jax.ShapeDtypeStruct((N_ATTRS, VOCAB, D_MODEL), jnp.float32),
    )(prior, option_embs, attr_emb.reshape(N_ATTRS, 1, D_MODEL))


def _idx_body(x_ref, out_ref):
    offs = lax.broadcasted_iota(jnp.int32, (BATCH, N_ATTRS), 1) * VOCAB
    out_ref[...] = x_ref[...] + offs


def _flat_idx(x):
    return pl.pallas_call(
        _idx_body,
        out_shape=jax.ShapeDtypeStruct((BATCH, N_ATTRS), jnp.int32),
    )(x)


SLICES = 4
SB = BATCH // SLICES
SROWS = SB * N_ATTRS
BPW = SB // NW
RPWS = BPW * N_ATTRS
CB = 16
CROWS = CB * N_ATTRS
GROWS = 104
GPC = CROWS // GROWS
NCHUNK = BPW // CB

_mesh = plsc.VectorSubcoreMesh(core_axis_name="c", subcore_axis_name="s")


@functools.partial(
    pl.kernel,
    mesh=_mesh,
    out_type=jax.ShapeDtypeStruct((SB, N_ATTRS, D_MODEL), jnp.float32),
    scratch_types=[
        pltpu.VMEM((RPWS,), jnp.int32),
        pltpu.VMEM((CROWS, D_MODEL), jnp.float32),
        pltpu.VMEM((CROWS, D_MODEL), jnp.float32),
        pltpu.SemaphoreType.DMA,
        pltpu.SemaphoreType.DMA,
        pltpu.SemaphoreType.DMA,
        pltpu.SemaphoreType.DMA,
    ],
)
def _gather_kernel(table_hbm, idx_hbm, out_hbm, idx_v, buf0, buf1,
                   g0, g1, s0, s1):
    wid = lax.axis_index("s") * NC + lax.axis_index("c")
    rbase = wid * RPWS
    bbase = wid * BPW
    pltpu.sync_copy(idx_hbm.at[pl.ds(rbase, RPWS)], idx_v)

    def start_gathers(c, buf, sem):
        for g in range(GPC):
            pltpu.async_copy(
                table_hbm.at[idx_v.at[pl.ds(c * CROWS + g * GROWS, GROWS)]],
                buf.at[pl.ds(g * GROWS, GROWS)], sem)

    def wait_gathers(buf, sem):
        pltpu.make_async_copy(table_hbm.at[pl.ds(0, CROWS)], buf, sem).wait()

    def start_put(c, buf, sem):
        pltpu.async_copy(buf.reshape(CB, N_ATTRS, D_MODEL),
                         out_hbm.at[pl.ds(bbase + c * CB, CB)], sem)

    def wait_put(c, buf, sem):
        pltpu.make_async_copy(buf.reshape(CB, N_ATTRS, D_MODEL),
                              out_hbm.at[pl.ds(bbase + c * CB, CB)],
                              sem).wait()

    start_gathers(0, buf0, g0)
    start_gathers(1, buf1, g1)

    def body(p, carry):
        c = 2 * p
        wait_gathers(buf0, g0)
        start_put(c, buf0, s0)
        wait_put(c, buf0, s0)
        start_gathers(c + 2, buf0, g0)
        wait_gathers(buf1, g1)
        start_put(c + 1, buf1, s1)
        wait_put(c + 1, buf1, s1)
        start_gathers(c + 3, buf1, g1)
        return carry

    lax.fori_loop(0, NCHUNK // 2 - 1, body, 0)

    c_last = NCHUNK - 2
    wait_gathers(buf0, g0)
    start_put(c_last, buf0, s0)
    wait_gathers(buf1, g1)
    start_put(c_last + 1, buf1, s1)
    wait_put(c_last, buf0, s0)
    wait_put(c_last + 1, buf1, s1)


def kernel(x, attr_emb, option_embs, prior):
    x = x.astype(jnp.int32)
    table = _fused_table(attr_emb, option_embs, prior).reshape(
        N_ATTRS * VOCAB, D_MODEL)
    idx = _flat_idx(x).reshape(ROWS)
    outs = [
        _gather_kernel(table, lax.slice(idx, (s * SROWS,), ((s + 1) * SROWS,)))
        for s in range(SLICES)
    ]
    return jnp.concatenate(outs, axis=0)


# speedup vs baseline: 11.7057x; 1.6469x over previous
import functools

import jax
import jax.numpy as jnp
from jax import lax
from jax.experimental import pallas as pl
from jax.experimental.pallas import tpu as pltpu
from jax.experimental.pallas import tpu_sc as plsc

N_ATTRS = 26
VOCAB = 1000
D_MODEL = 128
BATCH = 16384
ROWS = BATCH * N_ATTRS

NC = 2
NS = 16
NW = NC * NS
RPW = ROWS // NW
CHUNK = 128
NCH = RPW // CHUNK


def _fuse_body(prior_ref, opt_ref, attr_ref, out_ref):
    i = pl.program_id(0)
    out_ref[...] = (opt_ref[...] + attr_ref[...]) * prior_ref[i, 0]


def _fused_table(attr_emb, option_embs, prior):
    return pl.pallas_call(
        _fuse_body,
        grid=(N_ATTRS,),
        in_specs=[
            pl.BlockSpec(memory_space=pltpu.SMEM),
            pl.BlockSpec((1, VOCAB, D_MODEL), lambda i: (i, 0, 0)),
            pl.BlockSpec((1, 1, D_MODEL), lambda i: (i, 0, 0)),
        ],
        out_specs=pl.BlockSpec((1, VOCAB, D_MODEL), lambda i: (i, 0, 0)),
        out_shape=jax.ShapeDtypeStruct((N_ATTRS, VOCAB, D_MODEL), jnp.float32),
    )(prior, option_embs, attr_emb.reshape(N_ATTRS, 1, D_MODEL))


def _idx_body(x_ref, out_ref):
    offs = lax.broadcasted_iota(jnp.int32, (BATCH, N_ATTRS), 1) * VOCAB
    out_ref[...] = x_ref[...] + offs


def _flat_idx(x):
    return pl.pallas_call(
        _idx_body,
        out_shape=jax.ShapeDtypeStruct((BATCH, N_ATTRS), jnp.int32),
    )(x)


SLICES = 1
SB = BATCH // SLICES
SROWS = SB * N_ATTRS
BPW = SB // NW
RPWS = BPW * N_ATTRS
CB = 16
CROWS = CB * N_ATTRS
GROWS = 104
GPC = CROWS // GROWS
NCHUNK = BPW // CB

_mesh = plsc.VectorSubcoreMesh(core_axis_name="c", subcore_axis_name="s")


@functools.partial(
    pl.kernel,
    mesh=_mesh,
    out_type=jax.ShapeDtypeStruct((SB, N_ATTRS, D_MODEL), jnp.float32),
    compiler_params=pltpu.CompilerParams(use_tc_tiling_on_sc=True,
                                         needs_layout_passes=True),
    scratch_types=[
        pltpu.VMEM((RPWS,), jnp.int32),
        pltpu.VMEM((CROWS, D_MODEL), jnp.float32),
        pltpu.VMEM((CROWS, D_MODEL), jnp.float32),
        pltpu.SemaphoreType.DMA,
        pltpu.SemaphoreType.DMA,
        pltpu.SemaphoreType.DMA,
        pltpu.SemaphoreType.DMA,
    ],
)
def _gather_kernel(table_hbm, idx_hbm, out_hbm, idx_v, buf0, buf1,
                   g0, g1, s0, s1):
    wid = lax.axis_index("s") * NC + lax.axis_index("c")
    rbase = wid * RPWS
    bbase = wid * BPW
    pltpu.sync_copy(idx_hbm.at[pl.ds(rbase, RPWS)], idx_v)

    def start_gathers(c, buf, sem):
        for g in range(GPC):
            pltpu.async_copy(
                table_hbm.at[idx_v.at[pl.ds(c * CROWS + g * GROWS, GROWS)]],
                buf.at[pl.ds(g * GROWS, GROWS)], sem)

    def wait_gathers(buf, sem):
        pltpu.make_async_copy(table_hbm.at[pl.ds(0, CROWS)], buf, sem).wait()

    def start_put(c, buf, sem):
        pltpu.async_copy(buf.reshape(CB, N_ATTRS, D_MODEL),
                         out_hbm.at[pl.ds(bbase + c * CB, CB)], sem)

    def wait_put(c, buf, sem):
        pltpu.make_async_copy(buf.reshape(CB, N_ATTRS, D_MODEL),
                              out_hbm.at[pl.ds(bbase + c * CB, CB)],
                              sem).wait()

    start_gathers(0, buf0, g0)
    start_gathers(1, buf1, g1)

    def body(p, carry):
        c = 2 * p
        wait_gathers(buf0, g0)
        start_put(c, buf0, s0)
        wait_put(c, buf0, s0)
        start_gathers(c + 2, buf0, g0)
        wait_gathers(buf1, g1)
        start_put(c + 1, buf1, s1)
        wait_put(c + 1, buf1, s1)
        start_gathers(c + 3, buf1, g1)
        return carry

    lax.fori_loop(0, NCHUNK // 2 - 1, body, 0)

    c_last = NCHUNK - 2
    wait_gathers(buf0, g0)
    start_put(c_last, buf0, s0)
    wait_gathers(buf1, g1)
    start_put(c_last + 1, buf1, s1)
    wait_put(c_last, buf0, s0)
    wait_put(c_last + 1, buf1, s1)


def kernel(x, attr_emb, option_embs, prior):
    x = x.astype(jnp.int32)
    table = _fused_table(attr_emb, option_embs, prior).reshape(
        N_ATTRS * VOCAB, D_MODEL)
    idx = _flat_idx(x).reshape(ROWS)
    outs = [
        _gather_kernel(table, lax.slice(idx, (s * SROWS,), ((s + 1) * SROWS,)))
        for s in range(SLICES)
    ]
    return jnp.concatenate(outs, axis=0)
